# Initial kernel scaffold; baseline (speedup 1.0000x reference)
import jax
import jax.numpy as jnp
from jax.experimental import pallas as pl


def kernel(x, l1_w1, l1_b1, l1_w2, l1_b2, l1_g1, l1_be1, l1_m1, l1_v1, l1_g2, l1_be2, l1_m2, l1_v2, l2_w1, l2_b1, l2_w2, l2_b2, l2_g1, l2_be1, l2_m1, l2_v1, l2_g2, l2_be2, l2_m2, l2_v2, l3_w1, l3_b1, l3_w2, l3_b2, l3_g1, l3_be1, l3_m1, l3_v1, l3_g2, l3_be2, l3_m2, l3_v2, l4_w1, l4_b1, l4_w2, l4_b2, l4_g1, l4_be1, l4_m1, l4_v1, l4_g2, l4_be2, l4_m2, l4_v2, l5_w1, l5_b1, l5_w2, l5_b2, l5_g1, l5_be1, l5_m1, l5_v1, l5_g2, l5_be2, l5_m2, l5_v2, l6_w1, l6_b1, l6_w2, l6_b2, l6_g1, l6_be1, l6_m1, l6_v1, l6_g2, l6_be2, l6_m2, l6_v2, l7_w1, l7_b1, l7_w2, l7_b2, l7_g1, l7_be1, l7_m1, l7_v1, l7_g2, l7_be2, l7_m2, l7_v2, l8_w1, l8_b1, l8_w2, l8_b2, l8_g1, l8_be1, l8_m1, l8_v1, l8_g2, l8_be2, l8_m2, l8_v2, l9_w1, l9_b1, l9_w2, l9_b2, l9_g1, l9_be1, l9_m1, l9_v1, l9_g2, l9_be2, l9_m2, l9_v2, d1_w, d1_b, d1_g, d1_be, d1_m, d1_v, d2_w, d2_b, d2_g, d2_be, d2_m, d2_v, d3_w, d3_b, d3_g, d3_be, d3_m, d3_v, d4_w, d4_b, d4_g, d4_be, d4_m, d4_v, w10, b10):
    raise NotImplementedError("write your pallas kernel here")


# speedup vs baseline: 2.3219x; 2.3219x over previous
import functools

import jax
import jax.numpy as jnp
from jax import lax
from jax.experimental import pallas as pl
from jax.experimental.pallas import tpu as pltpu

_VMEM_LIMIT = 48 * 1024 * 1024


def _cp(dims):
    return pltpu.CompilerParams(dimension_semantics=dims,
                                vmem_limit_bytes=_VMEM_LIMIT)


def _fold_bn(gamma, beta, mean, var, eps=1e-5):
    scale = gamma / jnp.sqrt(var + eps)
    shift = beta - mean * scale
    return scale, shift


def _mask_cols(y3, w_valid):
    Wp, N = y3.shape[1], y3.shape[2]
    col = lax.broadcasted_iota(jnp.int32, (Wp, N), 0)
    valid = (col >= 1) & (col <= w_valid)
    return jnp.where(valid[None], y3, 0.0)


def _shift_slabs(F):
    z = jnp.zeros((1, F.shape[1]), F.dtype)
    down = jnp.concatenate([z, F[:-1]], axis=0)
    up = jnp.concatenate([F[1:], z], axis=0)
    return [down, F, up]


def _conv_fat_body(v_ref, w_ref, b_ref, o_ref, *, tr, Wp, C, w_valid, relu):
    M = tr * Wp
    slabs = []
    for dy in range(3):
        F = v_ref[dy].reshape(M, C)
        slabs.extend(_shift_slabs(F))
    lhs = jnp.concatenate(slabs, axis=1)
    y = jnp.dot(lhs, w_ref[...], preferred_element_type=jnp.float32)
    y = y + b_ref[...]
    if relu:
        y = jnp.maximum(y, 0.0)
    o_ref[...] = _mask_cols(y.reshape(tr, Wp, -1), w_valid)


def _conv_kdy_body(v_ref, w_ref, b_ref, o_ref, acc_ref, *, tr, Wp, C,
                   w_valid, relu):
    M = tr * Wp
    k = pl.program_id(1)
    F = v_ref[0].reshape(M, C)
    lhs = jnp.concatenate(_shift_slabs(F), axis=1)
    p = jnp.dot(lhs, w_ref[...], preferred_element_type=jnp.float32)

    @pl.when(k == 0)
    def _():
        acc_ref[...] = jnp.zeros_like(acc_ref)

    acc_ref[...] += p

    @pl.when(k == 2)
    def _():
        y = acc_ref[...] + b_ref[...]
        if relu:
            y = jnp.maximum(y, 0.0)
        o_ref[...] = _mask_cols(y.reshape(tr, Wp, -1), w_valid)


def _conv3x3(a, w, b, bn, *, tr, relu=True, kdy=False):
    B, H, Wp, C = a.shape
    w_valid = Wp - 8
    O = w.shape[0]
    w2 = jnp.transpose(w, (2, 3, 1, 0)).reshape(9 * C, O)
    if bn is not None:
        scale, shift = _fold_bn(*bn)
        w2 = w2 * scale[None, :]
        b2 = b * scale + shift
    else:
        b2 = b
    b2 = b2.reshape(1, O)

    xp = jnp.pad(a, ((0, 0), (1, 1), (0, 0), (0, 0)))
    V = jnp.stack([xp[:, d:d + H] for d in range(3)], axis=0)
    R = B * H
    V = V.reshape(3, R, Wp, C)
    nb = R // tr

    if not kdy:
        body = functools.partial(_conv_fat_body, tr=tr, Wp=Wp, C=C,
                                 w_valid=w_valid, relu=relu)
        out = pl.pallas_call(
            body,
            grid=(nb,),
            in_specs=[
                pl.BlockSpec((3, tr, Wp, C), lambda i: (0, i, 0, 0)),
                pl.BlockSpec((9 * C, O), lambda i: (0, 0)),
                pl.BlockSpec((1, O), lambda i: (0, 0)),
            ],
            out_specs=pl.BlockSpec((tr, Wp, O), lambda i: (i, 0, 0)),
            out_shape=jax.ShapeDtypeStruct((R, Wp, O), jnp.float32),
            compiler_params=_cp(("parallel",)),
        )(V, w2, b2)
    else:
        body = functools.partial(_conv_kdy_body, tr=tr, Wp=Wp, C=C,
                                 w_valid=w_valid, relu=relu)
        out = pl.pallas_call(
            body,
            grid=(nb, 3),
            in_specs=[
                pl.BlockSpec((1, tr, Wp, C), lambda i, k: (k, i, 0, 0)),
                pl.BlockSpec((3 * C, O), lambda i, k: (k, 0)),
                pl.BlockSpec((1, O), lambda i, k: (0, 0)),
            ],
            out_specs=pl.BlockSpec((tr, Wp, O), lambda i, k: (i, 0, 0)),
            out_shape=jax.ShapeDtypeStruct((R, Wp, O), jnp.float32),
            scratch_shapes=[pltpu.VMEM((tr * Wp, O), jnp.float32)],
            compiler_params=_cp(("parallel", "arbitrary")),
        )(V, w2, b2)
    return out.reshape(B, H, Wp, O)


def _gemm_body(x_ref, w_ref, b_ref, o_ref, *, relu, w_valid):
    tr, Wx, K = x_ref.shape
    y = jnp.dot(x_ref[...].reshape(tr * Wx, K), w_ref[...],
                preferred_element_type=jnp.float32)
    y = y + b_ref[...]
    if relu:
        y = jnp.maximum(y, 0.0)
    y3 = y.reshape(tr, Wx, -1)
    if w_valid is not None:
        y3 = _mask_cols(y3, w_valid)
    o_ref[...] = y3


def _gemm(x3, w2, b2, *, tr, relu, w_valid=None):
    R, Wx, K = x3.shape
    N = w2.shape[1]
    body = functools.partial(_gemm_body, relu=relu, w_valid=w_valid)
    return pl.pallas_call(
        body,
        grid=(R // tr,),
        in_specs=[
            pl.BlockSpec((tr, Wx, K), lambda i: (i, 0, 0)),
            pl.BlockSpec((K, N), lambda i: (0, 0)),
            pl.BlockSpec((1, N), lambda i: (0, 0)),
        ],
        out_specs=pl.BlockSpec((tr, Wx, N), lambda i: (i, 0, 0)),
        out_shape=jax.ShapeDtypeStruct((R, Wx, N), jnp.float32),
        compiler_params=_cp(("parallel",)),
    )(x3, w2, b2.reshape(1, N))


def _pool_body(a_ref, b_ref, c_ref, d_ref, o_ref):
    m = jnp.maximum(jnp.maximum(a_ref[...], b_ref[...]),
                    jnp.maximum(c_ref[...], d_ref[...]))
    tro, Wh, C = m.shape
    z1 = jnp.zeros((tro, 1, C), m.dtype)
    z7 = jnp.zeros((tro, 7, C), m.dtype)
    o_ref[...] = jnp.concatenate([z1, m, z7], axis=1)


def _pool(a, *, tr):
    B, H, Wp, C = a.shape
    Wv = Wp - 8
    Wh = Wv // 2
    Ro = B * H // 2
    slabs = [a[:, dy::2, 1 + dx:Wv + dx:2, :].reshape(Ro, Wh, C)
             for dy in (0, 1) for dx in (0, 1)]
    out = pl.pallas_call(
        _pool_body,
        grid=(Ro // tr,),
        in_specs=[pl.BlockSpec((tr, Wh, C), lambda i: (i, 0, 0))] * 4,
        out_specs=pl.BlockSpec((tr, Wh + 8, C), lambda i: (i, 0, 0)),
        out_shape=jax.ShapeDtypeStruct((Ro, Wh + 8, C), jnp.float32),
        compiler_params=_cp(("parallel",)),
    )(*slabs)
    return out.reshape(B, H // 2, Wh + 8, C)


def _deconv(a, w, b, bn, *, tr):
    B, h, wp, I = a.shape
    wv = wp - 8
    O = w.shape[1]
    scale, shift = _fold_bn(*bn)
    w2 = jnp.transpose(w, (0, 2, 3, 1)).reshape(I, 4 * O)
    w2 = w2 * jnp.tile(scale, 4)[None, :]
    b2 = jnp.tile(b * scale + shift, 4)
    G = _gemm(a.reshape(B * h, wp, I), w2, b2, tr=tr, relu=True)
    G6 = G.reshape(B, h, wp, 2, 2, O)[:, :, 1:wv + 1]
    t = jnp.transpose(G6, (0, 1, 3, 2, 4, 5)).reshape(B, 2 * h, 2 * wv, O)
    return jnp.pad(t, ((0, 0), (0, 0), (1, 7), (0, 0)))


def kernel(x, l1_w1, l1_b1, l1_w2, l1_b2, l1_g1, l1_be1, l1_m1, l1_v1, l1_g2, l1_be2, l1_m2, l1_v2, l2_w1, l2_b1, l2_w2, l2_b2, l2_g1, l2_be1, l2_m1, l2_v1, l2_g2, l2_be2, l2_m2, l2_v2, l3_w1, l3_b1, l3_w2, l3_b2, l3_g1, l3_be1, l3_m1, l3_v1, l3_g2, l3_be2, l3_m2, l3_v2, l4_w1, l4_b1, l4_w2, l4_b2, l4_g1, l4_be1, l4_m1, l4_v1, l4_g2, l4_be2, l4_m2, l4_v2, l5_w1, l5_b1, l5_w2, l5_b2, l5_g1, l5_be1, l5_m1, l5_v1, l5_g2, l5_be2, l5_m2, l5_v2, l6_w1, l6_b1, l6_w2, l6_b2, l6_g1, l6_be1, l6_m1, l6_v1, l6_g2, l6_be2, l6_m2, l6_v2, l7_w1, l7_b1, l7_w2, l7_b2, l7_g1, l7_be1, l7_m1, l7_v1, l7_g2, l7_be2, l7_m2, l7_v2, l8_w1, l8_b1, l8_w2, l8_b2, l8_g1, l8_be1, l8_m1, l8_v1, l8_g2, l8_be2, l8_m2, l8_v2, l9_w1, l9_b1, l9_w2, l9_b2, l9_g1, l9_be1, l9_m1, l9_v1, l9_g2, l9_be2, l9_m2, l9_v2, d1_w, d1_b, d1_g, d1_be, d1_m, d1_v, d2_w, d2_b, d2_g, d2_be, d2_m, d2_v, d3_w, d3_b, d3_g, d3_be, d3_m, d3_v, d4_w, d4_b, d4_g, d4_be, d4_m, d4_v, w10, b10):
    B, _, H, W = x.shape

    xn = jnp.transpose(x, (0, 2, 3, 1))
    xp2 = jnp.pad(xn, ((0, 0), (1, 1), (2, 8), (0, 0)))
    cols9 = jnp.concatenate(
        [xp2[:, dy:dy + H, dx:dx + (W + 8), :]
         for dy in range(3) for dx in range(3)], axis=-1)
    scale1, shift1 = _fold_bn(l1_g1, l1_be1, l1_m1, l1_v1)
    w2 = jnp.transpose(l1_w1, (2, 3, 1, 0)).reshape(27, 64) * scale1[None, :]
    b2 = l1_b1 * scale1 + shift1
    a = _gemm(cols9.reshape(B * H, W + 8, 27), w2, b2, tr=32, relu=True,
              w_valid=W)
    a = a.reshape(B, H, W + 8, 64)

    c1 = _conv3x3(a, l1_w2, l1_b2, (l1_g2, l1_be2, l1_m2, l1_v2), tr=32)

    a = _pool(c1, tr=32)
    a = _conv3x3(a, l2_w1, l2_b1, (l2_g1, l2_be1, l2_m1, l2_v1), tr=32)
    c2 = _conv3x3(a, l2_w2, l2_b2, (l2_g2, l2_be2, l2_m2, l2_v2), tr=32)

    a = _pool(c2, tr=32)
    a = _conv3x3(a, l3_w1, l3_b1, (l3_g1, l3_be1, l3_m1, l3_v1), tr=32)
    c3 = _conv3x3(a, l3_w2, l3_b2, (l3_g2, l3_be2, l3_m2, l3_v2), tr=32)

    a = _pool(c3, tr=16)
    a = _conv3x3(a, l4_w1, l4_b1, (l4_g1, l4_be1, l4_m1, l4_v1), tr=16)
    c4 = _conv3x3(a, l4_w2, l4_b2, (l4_g2, l4_be2, l4_m2, l4_v2), tr=16)

    a = _pool(c4, tr=16)
    a = _conv3x3(a, l5_w1, l5_b1, (l5_g1, l5_be1, l5_m1, l5_v1), tr=16,
                 kdy=True)
    c5 = _conv3x3(a, l5_w2, l5_b2, (l5_g2, l5_be2, l5_m2, l5_v2), tr=16,
                  kdy=True)

    t1 = _deconv(c5, d1_w, d1_b, (d1_g, d1_be, d1_m, d1_v), tr=16)
    a = jnp.concatenate([t1, c4], axis=-1)
    a = _conv3x3(a, l6_w1, l6_b1, (l6_g1, l6_be1, l6_m1, l6_v1), tr=32,
                 kdy=True)
    a = _conv3x3(a, l6_w2, l6_b2, (l6_g2, l6_be2, l6_m2, l6_v2), tr=16)

    t2 = _deconv(a, d2_w, d2_b, (d2_g, d2_be, d2_m, d2_v), tr=32)
    a = jnp.concatenate([t2, c3], axis=-1)
    a = _conv3x3(a, l7_w1, l7_b1, (l7_g1, l7_be1, l7_m1, l7_v1), tr=16)
    a = _conv3x3(a, l7_w2, l7_b2, (l7_g2, l7_be2, l7_m2, l7_v2), tr=32)

    t3 = _deconv(a, d3_w, d3_b, (d3_g, d3_be, d3_m, d3_v), tr=32)
    a = jnp.concatenate([t3, c2], axis=-1)
    a = _conv3x3(a, l8_w1, l8_b1, (l8_g1, l8_be1, l8_m1, l8_v1), tr=16)
    a = _conv3x3(a, l8_w2, l8_b2, (l8_g2, l8_be2, l8_m2, l8_v2), tr=32)

    t4 = _deconv(a, d4_w, d4_b, (d4_g, d4_be, d4_m, d4_v), tr=32)
    a = jnp.concatenate([t4, c1], axis=-1)
    a = _conv3x3(a, l9_w1, l9_b1, (l9_g1, l9_be1, l9_m1, l9_v1), tr=16)
    a = _conv3x3(a, l9_w2, l9_b2, (l9_g2, l9_be2, l9_m2, l9_v2), tr=32)

    y = _conv3x3(a, w10, b10, None, tr=32, relu=False)
    return jnp.transpose(y[:, :, 1:W + 1, :], (0, 3, 1, 2))


# speedup vs baseline: 4.4296x; 1.9077x over previous
import functools

import jax
import jax.numpy as jnp
from jax import lax
from jax.experimental import pallas as pl
from jax.experimental.pallas import tpu as pltpu

_VMEM_LIMIT = 48 * 1024 * 1024


def _cp(dims):
    return pltpu.CompilerParams(dimension_semantics=dims,
                                vmem_limit_bytes=_VMEM_LIMIT)


def _fold_bn(gamma, beta, mean, var, eps=1e-5):
    scale = gamma / jnp.sqrt(var + eps)
    shift = beta - mean * scale
    return scale, shift


def _mask_cols(y3, w_valid):
    Wp, N = y3.shape[1], y3.shape[2]
    col = lax.broadcasted_iota(jnp.int32, (Wp, N), 0)
    valid = (col >= 1) & (col <= w_valid)
    return jnp.where(valid[None], y3, 0.0)


def _halo_rows(prev_ref, cur_ref, next_ref, i, nbi):
    blk = i % nbi
    top = jnp.where(blk == 0, 0.0, prev_ref[-1:])
    bot = jnp.where(blk == nbi - 1, 0.0, next_ref[:1])
    return jnp.concatenate([top, cur_ref[...], bot], axis=0)


def _im2col_lhs(gf, tr, Wp, C):
    M = tr * Wp
    slabs = [gf[dy * Wp + dx:dy * Wp + dx + M]
             for dy in range(3) for dx in range(3)]
    return jnp.concatenate(slabs, axis=1)


def _pool_tile(y3, w_valid):
    tr, Wp, N = y3.shape
    z = y3.reshape(tr // 2, 2, Wp, N)
    return jnp.maximum(z[:, 0], z[:, 1])


def _colpool(rp):
    wv = rp.shape[2] - 8
    mp = jnp.maximum(rp[:, :, 1:wv + 1:2], rp[:, :, 2:wv + 2:2])
    return jnp.pad(mp, ((0, 0), (0, 0), (1, 7), (0, 0)))


def _conv_fat_body(prev_ref, cur_ref, next_ref, w_ref, b_ref, o_ref,
                   *maybe_pool_ref, tr, Wp, C, nbi, w_valid, relu, pool):
    g = _halo_rows(prev_ref, cur_ref, next_ref, pl.program_id(0), nbi)
    z1 = jnp.zeros((1, C), g.dtype)
    gf = jnp.concatenate([z1, g.reshape((tr + 2) * Wp, C), z1], axis=0)
    lhs = _im2col_lhs(gf, tr, Wp, C)
    y = jnp.dot(lhs, w_ref[...], preferred_element_type=jnp.float32)
    y = y + b_ref[...]
    if relu:
        y = jnp.maximum(y, 0.0)
    y3 = _mask_cols(y.reshape(tr, Wp, -1), w_valid)
    o_ref[...] = y3
    if pool:
        maybe_pool_ref[0][...] = _pool_tile(y3, w_valid)


def _conv_kdy_body(prev_ref, cur_ref, next_ref, w_ref, b_ref, o_ref,
                   g_ref, acc_ref, *, tr, Wp, C, nbi, w_valid, relu):
    M = tr * Wp
    i, k = pl.program_id(0), pl.program_id(1)

    @pl.when(k == 0)
    def _():
        g = _halo_rows(prev_ref, cur_ref, next_ref, i, nbi)
        z1 = jnp.zeros((1, C), g.dtype)
        z7 = jnp.zeros((7, C), g.dtype)
        g_ref[...] = jnp.concatenate(
            [z1, g.reshape((tr + 2) * Wp, C), z7], axis=0)
        acc_ref[...] = jnp.zeros_like(acc_ref)

    base = pl.multiple_of(k * Wp, 8)
    val = g_ref[pl.ds(base, M + 8)]
    lhs = jnp.concatenate([val[dx:dx + M] for dx in range(3)], axis=1)
    acc_ref[...] += jnp.dot(lhs, w_ref[...],
                            preferred_element_type=jnp.float32)

    @pl.when(k == 2)
    def _():
        y = acc_ref[...] + b_ref[...]
        if relu:
            y = jnp.maximum(y, 0.0)
        o_ref[...] = _mask_cols(y.reshape(tr, Wp, -1), w_valid)


def _conv3x3(a, w, b, bn, *, tr, relu=True, kdy=False, pool=False):
    B, H, Wp, C = a.shape
    w_valid = Wp - 8
    O = w.shape[0]
    w2 = jnp.transpose(w, (2, 3, 1, 0)).reshape(9 * C, O)
    if bn is not None:
        scale, shift = _fold_bn(*bn)
        w2 = w2 * scale[None, :]
        b2 = b * scale + shift
    else:
        b2 = b
    b2 = b2.reshape(1, O)

    R = B * H
    a2 = a.reshape(R, Wp, C)
    nb = R // tr
    nbi = H // tr

    def prev_map(i, *k):
        return (jnp.maximum(i - 1, 0), 0, 0)

    def cur_map(i, *k):
        return (i, 0, 0)

    def next_map(i, *k):
        return (jnp.minimum(i + 1, nb - 1), 0, 0)

    if not kdy:
        body = functools.partial(_conv_fat_body, tr=tr, Wp=Wp, C=C, nbi=nbi,
                                 w_valid=w_valid, relu=relu, pool=pool)
        out_specs = pl.BlockSpec((tr, Wp, O), lambda i: (i, 0, 0))
        out_shape = jax.ShapeDtypeStruct((R, Wp, O), jnp.float32)
        if pool:
            out_specs = [out_specs,
                         pl.BlockSpec((tr // 2, Wp, O), lambda i: (i, 0, 0))]
            out_shape = [out_shape,
                         jax.ShapeDtypeStruct((R // 2, Wp, O), jnp.float32)]
        out = pl.pallas_call(
            body,
            grid=(nb,),
            in_specs=[
                pl.BlockSpec((tr, Wp, C), prev_map),
                pl.BlockSpec((tr, Wp, C), cur_map),
                pl.BlockSpec((tr, Wp, C), next_map),
                pl.BlockSpec((9 * C, O), lambda i: (0, 0)),
                pl.BlockSpec((1, O), lambda i: (0, 0)),
            ],
            out_specs=out_specs,
            out_shape=out_shape,
            compiler_params=_cp(("parallel",)),
        )(a2, a2, a2, w2, b2)
        if pool:
            full, rowpooled = out
            return (full.reshape(B, H, Wp, O),
                    _colpool(rowpooled.reshape(B, H // 2, Wp, O)))
        return out.reshape(B, H, Wp, O)

    assert not pool
    body = functools.partial(_conv_kdy_body, tr=tr, Wp=Wp, C=C, nbi=nbi,
                             w_valid=w_valid, relu=relu)
    out = pl.pallas_call(
        body,
        grid=(nb, 3),
        in_specs=[
            pl.BlockSpec((tr, Wp, C), prev_map),
            pl.BlockSpec((tr, Wp, C), cur_map),
            pl.BlockSpec((tr, Wp, C), next_map),
            pl.BlockSpec((3 * C, O), lambda i, k: (k, 0)),
            pl.BlockSpec((1, O), lambda i, k: (0, 0)),
        ],
        out_specs=pl.BlockSpec((tr, Wp, O), lambda i, k: (i, 0, 0)),
        out_shape=jax.ShapeDtypeStruct((R, Wp, O), jnp.float32),
        scratch_shapes=[
            pltpu.VMEM((8 + (tr + 2) * Wp, C), jnp.float32),
            pltpu.VMEM((tr * Wp, O), jnp.float32),
        ],
        compiler_params=_cp(("parallel", "arbitrary")),
    )(a2, a2, a2, w2, b2)
    return out.reshape(B, H, Wp, O)


def _gemm_body(x_ref, w_ref, b_ref, o_ref, *, relu, w_valid):
    tr, Wx, K = x_ref.shape
    y = jnp.dot(x_ref[...].reshape(tr * Wx, K), w_ref[...],
                preferred_element_type=jnp.float32)
    y = y + b_ref[...]
    if relu:
        y = jnp.maximum(y, 0.0)
    y3 = y.reshape(tr, Wx, -1)
    if w_valid is not None:
        y3 = _mask_cols(y3, w_valid)
    o_ref[...] = y3


def _gemm(x3, w2, b2, *, tr, relu, w_valid=None):
    R, Wx, K = x3.shape
    N = w2.shape[1]
    body = functools.partial(_gemm_body, relu=relu, w_valid=w_valid)
    return pl.pallas_call(
        body,
        grid=(R // tr,),
        in_specs=[
            pl.BlockSpec((tr, Wx, K), lambda i: (i, 0, 0)),
            pl.BlockSpec((K, N), lambda i: (0, 0)),
            pl.BlockSpec((1, N), lambda i: (0, 0)),
        ],
        out_specs=pl.BlockSpec((tr, Wx, N), lambda i: (i, 0, 0)),
        out_shape=jax.ShapeDtypeStruct((R, Wx, N), jnp.float32),
        compiler_params=_cp(("parallel",)),
    )(x3, w2, b2.reshape(1, N))


def _deconv(a, w, b, bn, *, tr):
    B, h, wp, I = a.shape
    wv = wp - 8
    O = w.shape[1]
    scale, shift = _fold_bn(*bn)
    w2 = jnp.transpose(w, (0, 2, 3, 1)).reshape(I, 4 * O)
    w2 = w2 * jnp.tile(scale, 4)[None, :]
    b2 = jnp.tile(b * scale + shift, 4)
    G = _gemm(a.reshape(B * h, wp, I), w2, b2, tr=tr, relu=True)
    G6 = G.reshape(B, h, wp, 2, 2, O)[:, :, 1:wv + 1]
    t = jnp.transpose(G6, (0, 1, 3, 2, 4, 5)).reshape(B, 2 * h, 2 * wv, O)
    return jnp.pad(t, ((0, 0), (0, 0), (1, 7), (0, 0)))


def kernel(x, l1_w1, l1_b1, l1_w2, l1_b2, l1_g1, l1_be1, l1_m1, l1_v1, l1_g2, l1_be2, l1_m2, l1_v2, l2_w1, l2_b1, l2_w2, l2_b2, l2_g1, l2_be1, l2_m1, l2_v1, l2_g2, l2_be2, l2_m2, l2_v2, l3_w1, l3_b1, l3_w2, l3_b2, l3_g1, l3_be1, l3_m1, l3_v1, l3_g2, l3_be2, l3_m2, l3_v2, l4_w1, l4_b1, l4_w2, l4_b2, l4_g1, l4_be1, l4_m1, l4_v1, l4_g2, l4_be2, l4_m2, l4_v2, l5_w1, l5_b1, l5_w2, l5_b2, l5_g1, l5_be1, l5_m1, l5_v1, l5_g2, l5_be2, l5_m2, l5_v2, l6_w1, l6_b1, l6_w2, l6_b2, l6_g1, l6_be1, l6_m1, l6_v1, l6_g2, l6_be2, l6_m2, l6_v2, l7_w1, l7_b1, l7_w2, l7_b2, l7_g1, l7_be1, l7_m1, l7_v1, l7_g2, l7_be2, l7_m2, l7_v2, l8_w1, l8_b1, l8_w2, l8_b2, l8_g1, l8_be1, l8_m1, l8_v1, l8_g2, l8_be2, l8_m2, l8_v2, l9_w1, l9_b1, l9_w2, l9_b2, l9_g1, l9_be1, l9_m1, l9_v1, l9_g2, l9_be2, l9_m2, l9_v2, d1_w, d1_b, d1_g, d1_be, d1_m, d1_v, d2_w, d2_b, d2_g, d2_be, d2_m, d2_v, d3_w, d3_b, d3_g, d3_be, d3_m, d3_v, d4_w, d4_b, d4_g, d4_be, d4_m, d4_v, w10, b10):
    B, _, H, W = x.shape

    xn = jnp.transpose(x, (0, 2, 3, 1))
    xp2 = jnp.pad(xn, ((0, 0), (1, 1), (2, 8), (0, 0)))
    cols9 = jnp.concatenate(
        [xp2[:, dy:dy + H, dx:dx + (W + 8), :]
         for dy in range(3) for dx in range(3)], axis=-1)
    scale1, shift1 = _fold_bn(l1_g1, l1_be1, l1_m1, l1_v1)
    w2 = jnp.transpose(l1_w1, (2, 3, 1, 0)).reshape(27, 64) * scale1[None, :]
    b2 = l1_b1 * scale1 + shift1
    a = _gemm(cols9.reshape(B * H, W + 8, 27), w2, b2, tr=32, relu=True,
              w_valid=W)
    a = a.reshape(B, H, W + 8, 64)

    c1, a = _conv3x3(a, l1_w2, l1_b2, (l1_g2, l1_be2, l1_m2, l1_v2), tr=32,
                     pool=True)

    a = _conv3x3(a, l2_w1, l2_b1, (l2_g1, l2_be1, l2_m1, l2_v1), tr=32)
    c2, a = _conv3x3(a, l2_w2, l2_b2, (l2_g2, l2_be2, l2_m2, l2_v2), tr=32,
                     pool=True)

    a = _conv3x3(a, l3_w1, l3_b1, (l3_g1, l3_be1, l3_m1, l3_v1), tr=32)
    c3, a = _conv3x3(a, l3_w2, l3_b2, (l3_g2, l3_be2, l3_m2, l3_v2), tr=32,
                     pool=True)

    a = _conv3x3(a, l4_w1, l4_b1, (l4_g1, l4_be1, l4_m1, l4_v1), tr=16)
    c4, a = _conv3x3(a, l4_w2, l4_b2, (l4_g2, l4_be2, l4_m2, l4_v2), tr=16,
                     pool=True)

    a = _conv3x3(a, l5_w1, l5_b1, (l5_g1, l5_be1, l5_m1, l5_v1), tr=16,
                 kdy=True)
    c5 = _conv3x3(a, l5_w2, l5_b2, (l5_g2, l5_be2, l5_m2, l5_v2), tr=16,
                  kdy=True)

    t1 = _deconv(c5, d1_w, d1_b, (d1_g, d1_be, d1_m, d1_v), tr=16)
    a = jnp.concatenate([t1, c4], axis=-1)
    a = _conv3x3(a, l6_w1, l6_b1, (l6_g1, l6_be1, l6_m1, l6_v1), tr=32,
                 kdy=True)
    a = _conv3x3(a, l6_w2, l6_b2, (l6_g2, l6_be2, l6_m2, l6_v2), tr=16)

    t2 = _deconv(a, d2_w, d2_b, (d2_g, d2_be, d2_m, d2_v), tr=32)
    a = jnp.concatenate([t2, c3], axis=-1)
    a = _conv3x3(a, l7_w1, l7_b1, (l7_g1, l7_be1, l7_m1, l7_v1), tr=16)
    a = _conv3x3(a, l7_w2, l7_b2, (l7_g2, l7_be2, l7_m2, l7_v2), tr=32)

    t3 = _deconv(a, d3_w, d3_b, (d3_g, d3_be, d3_m, d3_v), tr=32)
    a = jnp.concatenate([t3, c2], axis=-1)
    a = _conv3x3(a, l8_w1, l8_b1, (l8_g1, l8_be1, l8_m1, l8_v1), tr=32)
    a = _conv3x3(a, l8_w2, l8_b2, (l8_g2, l8_be2, l8_m2, l8_v2), tr=32)

    t4 = _deconv(a, d4_w, d4_b, (d4_g, d4_be, d4_m, d4_v), tr=32)
    a = jnp.concatenate([t4, c1], axis=-1)
    a = _conv3x3(a, l9_w1, l9_b1, (l9_g1, l9_be1, l9_m1, l9_v1), tr=32)
    a = _conv3x3(a, l9_w2, l9_b2, (l9_g2, l9_be2, l9_m2, l9_v2), tr=32)

    y = _conv3x3(a, w10, b10, None, tr=32, relu=False)
    return jnp.transpose(y[:, :, 1:W + 1, :], (0, 3, 1, 2))


# speedup vs baseline: 4.5293x; 1.0225x over previous
import functools

import jax
import jax.numpy as jnp
from jax import lax
from jax.experimental import pallas as pl
from jax.experimental.pallas import tpu as pltpu

_VMEM_LIMIT = 48 * 1024 * 1024


def _cp(dims):
    return pltpu.CompilerParams(dimension_semantics=dims,
                                vmem_limit_bytes=_VMEM_LIMIT)


def _fold_bn(gamma, beta, mean, var, eps=1e-5):
    scale = gamma / jnp.sqrt(var + eps)
    shift = beta - mean * scale
    return scale, shift


def _mask_cols(y3, w_valid):
    Wp, N = y3.shape[1], y3.shape[2]
    col = lax.broadcasted_iota(jnp.int32, (Wp, N), 0)
    valid = (col >= 1) & (col <= w_valid)
    return jnp.where(valid[None], y3, 0.0)


def _halo_rows(prev_ref, cur_ref, next_ref, i, nbi):
    blk = i % nbi
    top = jnp.where(blk == 0, 0.0, prev_ref[-1:])
    bot = jnp.where(blk == nbi - 1, 0.0, next_ref[:1])
    return jnp.concatenate([top, cur_ref[...], bot], axis=0)


def _im2col_lhs(gf, tr, Wp, C):
    M = tr * Wp
    slabs = [gf[dy * Wp + dx:dy * Wp + dx + M]
             for dy in range(3) for dx in range(3)]
    return jnp.concatenate(slabs, axis=1)


def _pool_tile(y3, w_valid):
    tr, Wp, N = y3.shape
    z = y3.reshape(tr // 2, 2, Wp, N)
    return jnp.maximum(z[:, 0], z[:, 1])


def _colpool(rp):
    wv = rp.shape[2] - 8
    mp = jnp.maximum(rp[:, :, 1:wv + 1:2], rp[:, :, 2:wv + 2:2])
    return jnp.pad(mp, ((0, 0), (0, 0), (1, 7), (0, 0)))


def _guard_flat(prev_ref, cur_ref, next_ref, i, nbi, tr, Wp, C):
    g = _halo_rows(prev_ref, cur_ref, next_ref, i, nbi)
    z1 = jnp.zeros((1, C), g.dtype)
    return jnp.concatenate([z1, g.reshape((tr + 2) * Wp, C), z1], axis=0)


def _conv_fat_body(*refs, n_src, tr, Wp, nbi, w_valid, relu, pool):
    srcs = [refs[3 * s:3 * s + 3] for s in range(n_src)]
    w_ref, b_ref, o_ref = refs[3 * n_src:3 * n_src + 3]
    i = pl.program_id(0)
    M = tr * Wp
    gfs = [_guard_flat(*sr, i, nbi, tr, Wp, sr[0].shape[-1]) for sr in srcs]
    slabs = [gf[dy * Wp + dx:dy * Wp + dx + M]
             for dy in range(3) for dx in range(3) for gf in gfs]
    lhs = jnp.concatenate(slabs, axis=1)
    y = jnp.dot(lhs, w_ref[...], preferred_element_type=jnp.float32)
    y = y + b_ref[...]
    if relu:
        y = jnp.maximum(y, 0.0)
    y3 = _mask_cols(y.reshape(tr, Wp, -1), w_valid)
    o_ref[...] = y3
    if pool:
        refs[3 * n_src + 3][...] = _pool_tile(y3, w_valid)


def _conv_kdy_body(*refs, n_src, tr, Wp, nbi, w_valid, relu):
    srcs = [refs[3 * s:3 * s + 3] for s in range(n_src)]
    w_ref, b_ref, o_ref = refs[3 * n_src:3 * n_src + 3]
    g_refs = refs[3 * n_src + 3:3 * n_src + 3 + n_src]
    acc_ref = refs[3 * n_src + 3 + n_src]
    M = tr * Wp
    i, k = pl.program_id(0), pl.program_id(1)

    @pl.when(k == 0)
    def _():
        for sr, g_ref in zip(srcs, g_refs):
            C = sr[0].shape[-1]
            g = _halo_rows(*sr, i, nbi)
            z1 = jnp.zeros((1, C), g.dtype)
            z7 = jnp.zeros((7, C), g.dtype)
            g_ref[...] = jnp.concatenate(
                [z1, g.reshape((tr + 2) * Wp, C), z7], axis=0)
        acc_ref[...] = jnp.zeros_like(acc_ref)

    base = pl.multiple_of(k * Wp, 8)
    vals = [g_ref[pl.ds(base, M + 8)] for g_ref in g_refs]
    lhs = jnp.concatenate([v[dx:dx + M] for dx in range(3) for v in vals],
                          axis=1)
    acc_ref[...] += jnp.dot(lhs, w_ref[...],
                            preferred_element_type=jnp.float32)

    @pl.when(k == 2)
    def _():
        y = acc_ref[...] + b_ref[...]
        if relu:
            y = jnp.maximum(y, 0.0)
        o_ref[...] = _mask_cols(y.reshape(tr, Wp, -1), w_valid)


def _conv3x3(a, w, b, bn, *, tr, relu=True, kdy=False, pool=False):
    srcs = a if isinstance(a, (list, tuple)) else [a]
    B, H, Wp, _ = srcs[0].shape
    Cs = [s.shape[-1] for s in srcs]
    w_valid = Wp - 8
    O = w.shape[0]
    w2 = jnp.transpose(w, (2, 3, 1, 0)).reshape(9 * sum(Cs), O)
    if bn is not None:
        scale, shift = _fold_bn(*bn)
        w2 = w2 * scale[None, :]
        b2 = b * scale + shift
    else:
        b2 = b
    b2 = b2.reshape(1, O)

    R = B * H
    n_src = len(srcs)
    a2s = [s.reshape(R, Wp, C) for s, C in zip(srcs, Cs)]
    nb = R // tr
    nbi = H // tr

    def prev_map(i, *k):
        return (jnp.maximum(i - 1, 0), 0, 0)

    def cur_map(i, *k):
        return (i, 0, 0)

    def next_map(i, *k):
        return (jnp.minimum(i + 1, nb - 1), 0, 0)

    src_specs = []
    src_args = []
    for a2, C in zip(a2s, Cs):
        src_specs += [pl.BlockSpec((tr, Wp, C), m)
                      for m in (prev_map, cur_map, next_map)]
        src_args += [a2, a2, a2]

    if not kdy:
        body = functools.partial(_conv_fat_body, n_src=n_src, tr=tr, Wp=Wp,
                                 nbi=nbi, w_valid=w_valid, relu=relu,
                                 pool=pool)
        out_specs = pl.BlockSpec((tr, Wp, O), lambda i: (i, 0, 0))
        out_shape = jax.ShapeDtypeStruct((R, Wp, O), jnp.float32)
        if pool:
            out_specs = [out_specs,
                         pl.BlockSpec((tr // 2, Wp, O), lambda i: (i, 0, 0))]
            out_shape = [out_shape,
                         jax.ShapeDtypeStruct((R // 2, Wp, O), jnp.float32)]
        out = pl.pallas_call(
            body,
            grid=(nb,),
            in_specs=src_specs + [
                pl.BlockSpec((9 * sum(Cs), O), lambda i: (0, 0)),
                pl.BlockSpec((1, O), lambda i: (0, 0)),
            ],
            out_specs=out_specs,
            out_shape=out_shape,
            compiler_params=_cp(("parallel",)),
        )(*src_args, w2, b2)
        if pool:
            full, rowpooled = out
            return (full.reshape(B, H, Wp, O),
                    _colpool(rowpooled.reshape(B, H // 2, Wp, O)))
        return out.reshape(B, H, Wp, O)

    assert not pool
    body = functools.partial(_conv_kdy_body, n_src=n_src, tr=tr, Wp=Wp,
                             nbi=nbi, w_valid=w_valid, relu=relu)
    out = pl.pallas_call(
        body,
        grid=(nb, 3),
        in_specs=src_specs + [
            pl.BlockSpec((3 * sum(Cs), O), lambda i, k: (k, 0)),
            pl.BlockSpec((1, O), lambda i, k: (0, 0)),
        ],
        out_specs=pl.BlockSpec((tr, Wp, O), lambda i, k: (i, 0, 0)),
        out_shape=jax.ShapeDtypeStruct((R, Wp, O), jnp.float32),
        scratch_shapes=[pltpu.VMEM((8 + (tr + 2) * Wp, C), jnp.float32)
                        for C in Cs] +
                       [pltpu.VMEM((tr * Wp, O), jnp.float32)],
        compiler_params=_cp(("parallel", "arbitrary")),
    )(*src_args, w2, b2)
    return out.reshape(B, H, Wp, O)


def _gemm_body(x_ref, w_ref, b_ref, o_ref, *, relu, w_valid):
    tr, Wx, K = x_ref.shape
    y = jnp.dot(x_ref[...].reshape(tr * Wx, K), w_ref[...],
                preferred_element_type=jnp.float32)
    y = y + b_ref[...]
    if relu:
        y = jnp.maximum(y, 0.0)
    y3 = y.reshape(tr, Wx, -1)
    if w_valid is not None:
        y3 = _mask_cols(y3, w_valid)
    o_ref[...] = y3


def _gemm(x3, w2, b2, *, tr, relu, w_valid=None):
    R, Wx, K = x3.shape
    N = w2.shape[1]
    body = functools.partial(_gemm_body, relu=relu, w_valid=w_valid)
    return pl.pallas_call(
        body,
        grid=(R // tr,),
        in_specs=[
            pl.BlockSpec((tr, Wx, K), lambda i: (i, 0, 0)),
            pl.BlockSpec((K, N), lambda i: (0, 0)),
            pl.BlockSpec((1, N), lambda i: (0, 0)),
        ],
        out_specs=pl.BlockSpec((tr, Wx, N), lambda i: (i, 0, 0)),
        out_shape=jax.ShapeDtypeStruct((R, Wx, N), jnp.float32),
        compiler_params=_cp(("parallel",)),
    )(x3, w2, b2.reshape(1, N))


def _deconv(a, w, b, bn, *, tr):
    B, h, wp, I = a.shape
    wv = wp - 8
    O = w.shape[1]
    scale, shift = _fold_bn(*bn)
    w2 = jnp.transpose(w, (0, 2, 3, 1)).reshape(I, 4 * O)
    w2 = w2 * jnp.tile(scale, 4)[None, :]
    b2 = jnp.tile(b * scale + shift, 4)
    G = _gemm(a.reshape(B * h, wp, I), w2, b2, tr=tr, relu=True)
    G6 = G.reshape(B, h, wp, 2, 2, O)[:, :, 1:wv + 1]
    t = jnp.transpose(G6, (0, 1, 3, 2, 4, 5)).reshape(B, 2 * h, 2 * wv, O)
    return jnp.pad(t, ((0, 0), (0, 0), (1, 7), (0, 0)))


def kernel(x, l1_w1, l1_b1, l1_w2, l1_b2, l1_g1, l1_be1, l1_m1, l1_v1, l1_g2, l1_be2, l1_m2, l1_v2, l2_w1, l2_b1, l2_w2, l2_b2, l2_g1, l2_be1, l2_m1, l2_v1, l2_g2, l2_be2, l2_m2, l2_v2, l3_w1, l3_b1, l3_w2, l3_b2, l3_g1, l3_be1, l3_m1, l3_v1, l3_g2, l3_be2, l3_m2, l3_v2, l4_w1, l4_b1, l4_w2, l4_b2, l4_g1, l4_be1, l4_m1, l4_v1, l4_g2, l4_be2, l4_m2, l4_v2, l5_w1, l5_b1, l5_w2, l5_b2, l5_g1, l5_be1, l5_m1, l5_v1, l5_g2, l5_be2, l5_m2, l5_v2, l6_w1, l6_b1, l6_w2, l6_b2, l6_g1, l6_be1, l6_m1, l6_v1, l6_g2, l6_be2, l6_m2, l6_v2, l7_w1, l7_b1, l7_w2, l7_b2, l7_g1, l7_be1, l7_m1, l7_v1, l7_g2, l7_be2, l7_m2, l7_v2, l8_w1, l8_b1, l8_w2, l8_b2, l8_g1, l8_be1, l8_m1, l8_v1, l8_g2, l8_be2, l8_m2, l8_v2, l9_w1, l9_b1, l9_w2, l9_b2, l9_g1, l9_be1, l9_m1, l9_v1, l9_g2, l9_be2, l9_m2, l9_v2, d1_w, d1_b, d1_g, d1_be, d1_m, d1_v, d2_w, d2_b, d2_g, d2_be, d2_m, d2_v, d3_w, d3_b, d3_g, d3_be, d3_m, d3_v, d4_w, d4_b, d4_g, d4_be, d4_m, d4_v, w10, b10):
    B, _, H, W = x.shape

    xn = jnp.transpose(x, (0, 2, 3, 1))
    xp2 = jnp.pad(xn, ((0, 0), (1, 1), (2, 8), (0, 0)))
    cols9 = jnp.concatenate(
        [xp2[:, dy:dy + H, dx:dx + (W + 8), :]
         for dy in range(3) for dx in range(3)], axis=-1)
    scale1, shift1 = _fold_bn(l1_g1, l1_be1, l1_m1, l1_v1)
    w2 = jnp.transpose(l1_w1, (2, 3, 1, 0)).reshape(27, 64) * scale1[None, :]
    b2 = l1_b1 * scale1 + shift1
    a = _gemm(cols9.reshape(B * H, W + 8, 27), w2, b2, tr=32, relu=True,
              w_valid=W)
    a = a.reshape(B, H, W + 8, 64)

    c1, a = _conv3x3(a, l1_w2, l1_b2, (l1_g2, l1_be2, l1_m2, l1_v2), tr=32,
                     pool=True)

    a = _conv3x3(a, l2_w1, l2_b1, (l2_g1, l2_be1, l2_m1, l2_v1), tr=32)
    c2, a = _conv3x3(a, l2_w2, l2_b2, (l2_g2, l2_be2, l2_m2, l2_v2), tr=32,
                     pool=True)

    a = _conv3x3(a, l3_w1, l3_b1, (l3_g1, l3_be1, l3_m1, l3_v1), tr=32)
    c3, a = _conv3x3(a, l3_w2, l3_b2, (l3_g2, l3_be2, l3_m2, l3_v2), tr=32,
                     pool=True)

    a = _conv3x3(a, l4_w1, l4_b1, (l4_g1, l4_be1, l4_m1, l4_v1), tr=16)
    c4, a = _conv3x3(a, l4_w2, l4_b2, (l4_g2, l4_be2, l4_m2, l4_v2), tr=16,
                     pool=True)

    a = _conv3x3(a, l5_w1, l5_b1, (l5_g1, l5_be1, l5_m1, l5_v1), tr=16,
                 kdy=True)
    c5 = _conv3x3(a, l5_w2, l5_b2, (l5_g2, l5_be2, l5_m2, l5_v2), tr=16,
                  kdy=True)

    t1 = _deconv(c5, d1_w, d1_b, (d1_g, d1_be, d1_m, d1_v), tr=16)
    a = _conv3x3([t1, c4], l6_w1, l6_b1, (l6_g1, l6_be1, l6_m1, l6_v1), tr=32,
                 kdy=True)
    a = _conv3x3(a, l6_w2, l6_b2, (l6_g2, l6_be2, l6_m2, l6_v2), tr=16)

    t2 = _deconv(a, d2_w, d2_b, (d2_g, d2_be, d2_m, d2_v), tr=32)
    a = _conv3x3([t2, c3], l7_w1, l7_b1, (l7_g1, l7_be1, l7_m1, l7_v1), tr=16)
    a = _conv3x3(a, l7_w2, l7_b2, (l7_g2, l7_be2, l7_m2, l7_v2), tr=32)

    t3 = _deconv(a, d3_w, d3_b, (d3_g, d3_be, d3_m, d3_v), tr=32)
    a = _conv3x3([t3, c2], l8_w1, l8_b1, (l8_g1, l8_be1, l8_m1, l8_v1), tr=32)
    a = _conv3x3(a, l8_w2, l8_b2, (l8_g2, l8_be2, l8_m2, l8_v2), tr=32)

    t4 = _deconv(a, d4_w, d4_b, (d4_g, d4_be, d4_m, d4_v), tr=32)
    a = _conv3x3([t4, c1], l9_w1, l9_b1, (l9_g1, l9_be1, l9_m1, l9_v1), tr=32)
    a = _conv3x3(a, l9_w2, l9_b2, (l9_g2, l9_be2, l9_m2, l9_v2), tr=32)

    y = _conv3x3(a, w10, b10, None, tr=32, relu=False)
    return jnp.transpose(y[:, :, 1:W + 1, :], (0, 3, 1, 2))
